# baseline (speedup 1.0000x reference)
import functools

import jax
import jax.numpy as jnp
from jax import lax
from jax.experimental import pallas as pl
from jax.experimental.pallas import tpu as pltpu
from jax.experimental.pallas import tpu_sc as plsc

_N_SCORE = 5000
_N_NMS = 1000
_TH = 0.7
_TILE = 128
_NT = 40
_NPAD = _NT * _TILE
_OUT_PAD = 1024

_SORT_N = 32768
_SORT_R = _SORT_N // 128
_N_IN = 20000


def _sort_body(keys_ref, outidx_ref):
    def flat_of(rows):
        r = lax.broadcasted_iota(jnp.int32, (rows, 128), 0)
        c = lax.broadcasted_iota(jnp.int32, (rows, 128), 1)
        return r * 128 + c

    def resolve(keys, idx, pkey, pidx, bit, k, flat):
        first = (keys > pkey) | ((keys == pkey) & (idx < pidx))
        dir_asc = (flat & k) != 0
        keep_mine = (first != bit) != dir_asc
        return (jnp.where(keep_mine, keys, pkey),
                jnp.where(keep_mine, idx, pidx))

    def ex_sub(keys, idx, k, j, flat, rows):
        js = lax.shift_right_logical(j, 7)
        ku = pltpu.roll(keys, rows - js, axis=0)
        kd = pltpu.roll(keys, js, axis=0)
        iu = pltpu.roll(idx, rows - js, axis=0)
        id_ = pltpu.roll(idx, js, axis=0)
        bit = (flat & j) != 0
        return resolve(keys, idx, jnp.where(bit, kd, ku),
                       jnp.where(bit, id_, iu), bit, k, flat)

    def ex_lane(keys, idx, k, j, flat):
        ku = pltpu.roll(keys, 128 - j, axis=1)
        kd = pltpu.roll(keys, j, axis=1)
        iu = pltpu.roll(idx, 128 - j, axis=1)
        id_ = pltpu.roll(idx, j, axis=1)
        bit = (flat & j) != 0
        return resolve(keys, idx, jnp.where(bit, kd, ku),
                       jnp.where(bit, id_, iu), bit, k, flat)

    def merge_pass(keys, idx, k, j0, flat, rows):
        def isub(s):
            keys, idx, j = s
            keys, idx = ex_sub(keys, idx, k, j, flat, rows)
            return keys, idx, lax.shift_right_logical(j, 1)

        def ilane(s):
            keys, idx, j = s
            keys, idx = ex_lane(keys, idx, k, j, flat)
            return keys, idx, lax.shift_right_logical(j, 1)

        s = lax.while_loop(lambda s: s[2] >= 128, isub, (keys, idx, j0))
        keys, idx, _ = lax.while_loop(lambda s: s[2] >= 1, ilane, s)
        return keys, idx

    rows_a = 160
    flat_a = flat_of(rows_a)
    keys_a = keys_ref[0][:rows_a]
    idx_a = flat_a

    def pass_a(s):
        keys, idx, k = s
        keys, idx = merge_pass(keys, idx, k, lax.shift_right_logical(k, 1),
                               flat_a, rows_a)
        return keys, idx, k * 2

    keys_a, idx_a, _ = lax.while_loop(lambda s: s[2] <= 4096, pass_a,
                                      (keys_a, idx_a, jnp.int32(2)))

    flat_b = flat_of(_SORT_R)
    keys = jnp.concatenate([keys_a, keys_ref[0][rows_a:]], axis=0)
    idx = jnp.concatenate([idx_a, flat_b[rows_a:]], axis=0)
    for k in (8192, 16384):
        keys, idx = merge_pass(keys, idx, jnp.int32(k), jnp.int32(k // 2),
                               flat_b, _SORT_R)
    k = jnp.int32(_SORT_N)
    keys, idx = ex_sub(keys, idx, k, jnp.int32(16384), flat_b, _SORT_R)
    keys, idx = keys[:128], idx[:128]
    keys, idx = ex_sub(keys, idx, k, jnp.int32(8192), flat_of(128), 128)
    keys, idx = keys[:64], idx[:64]
    keys, idx = merge_pass(keys, idx, k, jnp.int32(4096), flat_of(64), 64)
    outidx_ref[0] = idx[:_NT, :]


def _sort_call(keys):
    b = keys.shape[0]
    return pl.pallas_call(
        _sort_body,
        grid=(b,),
        in_specs=[pl.BlockSpec((1, _SORT_R, 128), lambda bb: (bb, 0, 0))],
        out_specs=pl.BlockSpec((1, _NT, 128), lambda bb: (bb, 0, 0)),
        out_shape=jax.ShapeDtypeStruct((b, _NT, 128), jnp.int32),
        compiler_params=pltpu.CompilerParams(
            dimension_semantics=("parallel",)),
    )(keys)


_GCHUNK = (4 * _NPAD) // 32
_IMG_WORDS = _N_IN * 4


def _gather_body(idx_hbm, roi_hbm, out_hbm, idx_v, table_v, rows_v, sem):
    wid = lax.axis_index("s") * 2 + lax.axis_index("c")
    base = wid * _GCHUNK
    img = wid // 8
    pltpu.sync_copy(idx_hbm.at[pl.ds(base, _GCHUNK)], idx_v)
    pltpu.async_copy(roi_hbm.at[pl.ds(img * _IMG_WORDS, _IMG_WORDS)],
                     table_v, sem).wait()
    lane = lax.broadcasted_iota(jnp.int32, (16,), 0)

    def group(g, _):
        i16 = idx_v[pl.ds(g * 16, 16)]
        for c in range(4):
            vals = plsc.load_gather(table_v, [i16 * 4 + c])
            plsc.store_scatter(rows_v, [g * 64 + lane * 4 + c], vals)
        return 0

    lax.fori_loop(0, _GCHUNK // 16, group, 0)
    pltpu.sync_copy(rows_v, out_hbm.at[pl.ds(base * 4, _GCHUNK * 4)])


def _gather_call(gidx, roi_flat):
    n = gidx.shape[0]
    kern = functools.partial(
        pl.kernel,
        mesh=plsc.VectorSubcoreMesh(core_axis_name="c", subcore_axis_name="s"),
        out_type=jax.ShapeDtypeStruct((n * 4,), jnp.float32),
        scratch_types=[
            pltpu.VMEM((_GCHUNK,), jnp.int32),
            pltpu.VMEM((_IMG_WORDS,), jnp.float32),
            pltpu.VMEM((_GCHUNK * 4,), jnp.float32),
            pltpu.SemaphoreType.DMA,
        ],
        compiler_params=pltpu.CompilerParams(use_tc_tiling_on_sc=False,
                                             needs_layout_passes=False),
    )(_gather_body)
    return kern(gidx, roi_flat).reshape(n, 4)


def _nms_body(boxes_ref, boxesT_ref, out_ref, keep_ref):
    out_ref[0] = jnp.zeros((_OUT_PAD, 4), jnp.float32)

    iota_r = lax.broadcasted_iota(jnp.int32, (_TILE, _TILE), 0)
    iota_c = lax.broadcasted_iota(jnp.int32, (_TILE, _TILE), 1)
    lower_tri = jnp.where(iota_r > iota_c, 1.0, 0.0)
    incl_tri = jnp.where(iota_r <= iota_c, 1.0, 0.0)
    eye = jnp.where(iota_r == iota_c, 1.0, 0.0)
    col_iota = lax.broadcasted_iota(jnp.int32, (_TILE, 1), 0)
    out_iota = lax.broadcasted_iota(jnp.int32, (_OUT_PAD, 1), 0)

    def iou_vs(y1r, x1r, y2r, x2r, area_r, tile_t):
        y1c = tile_t[0:1, :]
        x1c = tile_t[1:2, :]
        y2c = tile_t[2:3, :]
        x2c = tile_t[3:4, :]
        area_c = (y2c - y1c) * (x2c - x1c)
        ih = jnp.maximum(0.0, jnp.minimum(y2r, y2c) - jnp.maximum(y1r, y1c))
        iw = jnp.maximum(0.0, jnp.minimum(x2r, x2c) - jnp.maximum(x1r, x1c))
        inter = ih * iw
        union = area_r + area_c - inter
        return jnp.where(union > 0.0, inter / union, 0.0)

    def process_tile(carry):
        i, cnt = carry
        tile = boxes_ref[0, i]
        tile_t = boxesT_ref[0, i]
        y1r = tile[:, 0:1]
        x1r = tile[:, 1:2]
        y2r = tile[:, 2:3]
        x2r = tile[:, 3:4]
        area_r = (y2r - y1r) * (x2r - x1r)

        def cross(j, sup):
            iou = iou_vs(y1r, x1r, y2r, x2r, area_r, boxesT_ref[0, j])
            keep_j = keep_ref[j, 0:1, :]
            hit = jnp.where((iou > _TH) & (keep_j > 0.0), 1.0, 0.0)
            return sup + jnp.sum(hit, axis=1, keepdims=True)

        sup = lax.fori_loop(0, i, cross, jnp.zeros((_TILE, 1), jnp.float32))

        iou_self = iou_vs(y1r, x1r, y2r, x2r, area_r, tile_t)
        m_mat = jnp.where(iou_self > _TH, 1.0, 0.0) * lower_tri
        valid = jnp.where(i * _TILE + col_iota < _N_SCORE, 1.0, 0.0)
        base = valid * jnp.where(sup > 0.0, 0.0, 1.0)

        def fix_step(k):
            s = jnp.dot(m_mat, k, preferred_element_type=jnp.float32)
            return base * jnp.where(s > 0.0, 0.0, 1.0)

        def fix_cond(c):
            old, new = c
            return jnp.any(old != new)

        def fix_body(c):
            _, k = c
            return k, fix_step(k)

        _, keep = lax.while_loop(fix_cond, fix_body, (base, fix_step(base)))

        keep_row = jnp.sum(keep * eye, axis=0, keepdims=True)
        keep_ref[i, 0:1, :] = keep_row

        cum = jnp.dot(keep_row, incl_tri, preferred_element_type=jnp.float32)
        pos = cnt + cum.astype(jnp.int32) - 1
        onehot = jnp.where((out_iota == pos) & (keep_row > 0.0), 1.0, 0.0)
        cols = [
            jnp.sum(onehot * tile_t[c : c + 1, :], axis=1, keepdims=True)
            for c in range(4)
        ]
        out_ref[0] += jnp.concatenate(cols, axis=1)

        new_cnt = cnt + jnp.sum(keep_row).astype(jnp.int32)
        return i + 1, new_cnt

    def outer_cond(carry):
        i, cnt = carry
        return (i < _NT) & (cnt < _N_NMS)

    lax.while_loop(outer_cond, process_tile, (jnp.int32(0), jnp.int32(0)))


def _nms_call(boxes, boxes_t):
    b = boxes.shape[0]
    return pl.pallas_call(
        _nms_body,
        grid=(b,),
        in_specs=[
            pl.BlockSpec((1, _NT, _TILE, 4), lambda bb: (bb, 0, 0, 0)),
            pl.BlockSpec((1, _NT, 4, _TILE), lambda bb: (bb, 0, 0, 0)),
        ],
        out_specs=pl.BlockSpec((1, _OUT_PAD, 4), lambda bb: (bb, 0, 0)),
        out_shape=jax.ShapeDtypeStruct((b, _OUT_PAD, 4), jnp.float32),
        scratch_shapes=[pltpu.VMEM((_NT, 8, _TILE), jnp.float32)],
        compiler_params=pltpu.CompilerParams(
            dimension_semantics=("parallel",)),
    )(boxes, boxes_t)


def kernel(rpn_prob, rpn_roi):
    b = rpn_prob.shape[0]
    n = rpn_prob.shape[1]
    scores = rpn_prob[..., 0]
    bits = lax.bitcast_convert_type(scores, jnp.int32)
    keys = jnp.concatenate(
        [bits, jnp.full((b, _SORT_N - n), -1, jnp.int32)], axis=1)
    sidx = _sort_call(keys.reshape(b, _SORT_R, 128))
    gidx = sidx.reshape(b * _NPAD)
    top_roi = _gather_call(gidx, rpn_roi.reshape(b * n * 4))
    boxes = top_roi.reshape(b, _NT, _TILE, 4)
    boxes_t = jnp.swapaxes(boxes, 2, 3)
    out = _nms_call(boxes, boxes_t)
    return out[:, :_N_NMS, :]


# speedup vs baseline: 1.1500x; 1.1500x over previous
import functools

import jax
import jax.numpy as jnp
from jax import lax
from jax.experimental import pallas as pl
from jax.experimental.pallas import tpu as pltpu
from jax.experimental.pallas import tpu_sc as plsc

_N_SCORE = 5000
_N_NMS = 1000
_TH = 0.7
_TILE = 256
_NT = 20
_NPAD = _NT * _TILE
_OUT_PAD = 1024

_SORT_N = 32768
_SORT_R = _SORT_N // 128
_SIDX_R = _NPAD // 128
_N_IN = 20000


def _sort_body(keys_ref, outidx_ref):
    def flat_of(rows):
        r = lax.broadcasted_iota(jnp.int32, (rows, 128), 0)
        c = lax.broadcasted_iota(jnp.int32, (rows, 128), 1)
        return r * 128 + c

    def resolve(keys, idx, pkey, pidx, bit, k, flat):
        first = (keys > pkey) | ((keys == pkey) & (idx < pidx))
        dir_asc = (flat & k) != 0
        keep_mine = (first != bit) != dir_asc
        return (jnp.where(keep_mine, keys, pkey),
                jnp.where(keep_mine, idx, pidx))

    def ex_sub(keys, idx, k, j, flat, rows):
        js = lax.shift_right_logical(j, 7)
        ku = pltpu.roll(keys, rows - js, axis=0)
        kd = pltpu.roll(keys, js, axis=0)
        iu = pltpu.roll(idx, rows - js, axis=0)
        id_ = pltpu.roll(idx, js, axis=0)
        bit = (flat & j) != 0
        return resolve(keys, idx, jnp.where(bit, kd, ku),
                       jnp.where(bit, id_, iu), bit, k, flat)

    def ex_lane(keys, idx, k, j, flat):
        ku = pltpu.roll(keys, 128 - j, axis=1)
        kd = pltpu.roll(keys, j, axis=1)
        iu = pltpu.roll(idx, 128 - j, axis=1)
        id_ = pltpu.roll(idx, j, axis=1)
        bit = (flat & j) != 0
        return resolve(keys, idx, jnp.where(bit, kd, ku),
                       jnp.where(bit, id_, iu), bit, k, flat)

    def merge_pass(keys, idx, k, j0, flat, rows):
        def isub(s):
            keys, idx, j = s
            keys, idx = ex_sub(keys, idx, k, j, flat, rows)
            return keys, idx, lax.shift_right_logical(j, 1)

        def ilane(s):
            keys, idx, j = s
            keys, idx = ex_lane(keys, idx, k, j, flat)
            return keys, idx, lax.shift_right_logical(j, 1)

        s = lax.while_loop(lambda s: s[2] >= 128, isub, (keys, idx, j0))
        keys, idx, _ = lax.while_loop(lambda s: s[2] >= 1, ilane, s)
        return keys, idx

    rows_a = 160
    flat_a = flat_of(rows_a)
    keys_a = keys_ref[0][:rows_a]
    idx_a = flat_a

    def pass_a(s):
        keys, idx, k = s
        keys, idx = merge_pass(keys, idx, k, lax.shift_right_logical(k, 1),
                               flat_a, rows_a)
        return keys, idx, k * 2

    keys_a, idx_a, _ = lax.while_loop(lambda s: s[2] <= 4096, pass_a,
                                      (keys_a, idx_a, jnp.int32(2)))

    flat_b = flat_of(_SORT_R)
    keys = jnp.concatenate([keys_a, keys_ref[0][rows_a:]], axis=0)
    idx = jnp.concatenate([idx_a, flat_b[rows_a:]], axis=0)
    for k in (8192, 16384):
        keys, idx = merge_pass(keys, idx, jnp.int32(k), jnp.int32(k // 2),
                               flat_b, _SORT_R)
    k = jnp.int32(_SORT_N)
    keys, idx = ex_sub(keys, idx, k, jnp.int32(16384), flat_b, _SORT_R)
    keys, idx = keys[:128], idx[:128]
    keys, idx = ex_sub(keys, idx, k, jnp.int32(8192), flat_of(128), 128)
    keys, idx = keys[:64], idx[:64]
    keys, idx = merge_pass(keys, idx, k, jnp.int32(4096), flat_of(64), 64)
    outidx_ref[0] = idx[:_SIDX_R, :]


def _sort_call(keys):
    b = keys.shape[0]
    return pl.pallas_call(
        _sort_body,
        grid=(b,),
        in_specs=[pl.BlockSpec((1, _SORT_R, 128), lambda bb: (bb, 0, 0))],
        out_specs=pl.BlockSpec((1, _SIDX_R, 128), lambda bb: (bb, 0, 0)),
        out_shape=jax.ShapeDtypeStruct((b, _SIDX_R, 128), jnp.int32),
        compiler_params=pltpu.CompilerParams(
            dimension_semantics=("parallel",)),
    )(keys)


_GCHUNK = (4 * _NPAD) // 32
_IMG_WORDS = _N_IN * 4


def _gather_body(idx_hbm, roi_hbm, out_hbm, idx_v, table_v, rows_v, sem):
    wid = lax.axis_index("s") * 2 + lax.axis_index("c")
    base = wid * _GCHUNK
    img = wid // 8
    pltpu.sync_copy(idx_hbm.at[pl.ds(base, _GCHUNK)], idx_v)
    pltpu.async_copy(roi_hbm.at[pl.ds(img * _IMG_WORDS, _IMG_WORDS)],
                     table_v, sem).wait()
    lane = lax.broadcasted_iota(jnp.int32, (16,), 0)

    def group(g, _):
        i16 = idx_v[pl.ds(g * 16, 16)]
        for c in range(4):
            vals = plsc.load_gather(table_v, [i16 * 4 + c])
            plsc.store_scatter(rows_v, [g * 64 + lane * 4 + c], vals)
        return 0

    lax.fori_loop(0, _GCHUNK // 16, group, 0)
    pltpu.sync_copy(rows_v, out_hbm.at[pl.ds(base * 4, _GCHUNK * 4)])


def _gather_call(gidx, roi_flat):
    n = gidx.shape[0]
    kern = functools.partial(
        pl.kernel,
        mesh=plsc.VectorSubcoreMesh(core_axis_name="c", subcore_axis_name="s"),
        out_type=jax.ShapeDtypeStruct((n * 4,), jnp.float32),
        scratch_types=[
            pltpu.VMEM((_GCHUNK,), jnp.int32),
            pltpu.VMEM((_IMG_WORDS,), jnp.float32),
            pltpu.VMEM((_GCHUNK * 4,), jnp.float32),
            pltpu.SemaphoreType.DMA,
        ],
        compiler_params=pltpu.CompilerParams(use_tc_tiling_on_sc=False,
                                             needs_layout_passes=False),
    )(_gather_body)
    return kern(gidx, roi_flat).reshape(n, 4)


def _nms_body(boxes_ref, boxesT_ref, out_ref, keep_ref):
    out_ref[0] = jnp.zeros((_OUT_PAD, 4), jnp.float32)

    iota_r = lax.broadcasted_iota(jnp.int32, (_TILE, _TILE), 0)
    iota_c = lax.broadcasted_iota(jnp.int32, (_TILE, _TILE), 1)
    lower_tri = jnp.where(iota_r > iota_c, 1.0, 0.0)
    incl_tri = jnp.where(iota_r <= iota_c, 1.0, 0.0)
    eye = jnp.where(iota_r == iota_c, 1.0, 0.0)
    col_iota = lax.broadcasted_iota(jnp.int32, (_TILE, 1), 0)
    out_iota = lax.broadcasted_iota(jnp.int32, (_OUT_PAD, 1), 0)

    def iou_vs(y1r, x1r, y2r, x2r, area_r, tile_t):
        y1c = tile_t[0:1, :]
        x1c = tile_t[1:2, :]
        y2c = tile_t[2:3, :]
        x2c = tile_t[3:4, :]
        area_c = (y2c - y1c) * (x2c - x1c)
        ih = jnp.maximum(0.0, jnp.minimum(y2r, y2c) - jnp.maximum(y1r, y1c))
        iw = jnp.maximum(0.0, jnp.minimum(x2r, x2c) - jnp.maximum(x1r, x1c))
        inter = ih * iw
        union = area_r + area_c - inter
        return jnp.where(union > 0.0, inter / union, 0.0)

    def process_tile(carry):
        i, cnt = carry
        tile = boxes_ref[0, i]
        tile_t = boxesT_ref[0, i]
        y1r = tile[:, 0:1]
        x1r = tile[:, 1:2]
        y2r = tile[:, 2:3]
        x2r = tile[:, 3:4]
        area_r = (y2r - y1r) * (x2r - x1r)

        def cross(j, sup):
            iou = iou_vs(y1r, x1r, y2r, x2r, area_r, boxesT_ref[0, j])
            keep_j = keep_ref[j, 0:1, :]
            hit = jnp.where((iou > _TH) & (keep_j > 0.0), 1.0, 0.0)
            return sup + jnp.sum(hit, axis=1, keepdims=True)

        sup = lax.fori_loop(0, i, cross, jnp.zeros((_TILE, 1), jnp.float32))

        iou_self = iou_vs(y1r, x1r, y2r, x2r, area_r, tile_t)
        m_mat = jnp.where(iou_self > _TH, 1.0, 0.0) * lower_tri
        valid = jnp.where(i * _TILE + col_iota < _N_SCORE, 1.0, 0.0)
        base = valid * jnp.where(sup > 0.0, 0.0, 1.0)

        def fix_step(k):
            s = jnp.dot(m_mat, k, preferred_element_type=jnp.float32)
            return base * jnp.where(s > 0.0, 0.0, 1.0)

        def fix_cond(c):
            old, new = c
            return jnp.any(old != new)

        def fix_body(c):
            _, k = c
            return k, fix_step(k)

        _, keep = lax.while_loop(fix_cond, fix_body, (base, fix_step(base)))

        keep_row = jnp.sum(keep * eye, axis=0, keepdims=True)
        keep_ref[i, 0:1, :] = keep_row

        cum = jnp.dot(keep_row, incl_tri, preferred_element_type=jnp.float32)
        pos = cnt + cum.astype(jnp.int32) - 1
        onehot = jnp.where((out_iota == pos) & (keep_row > 0.0), 1.0, 0.0)
        cols = [
            jnp.sum(onehot * tile_t[c : c + 1, :], axis=1, keepdims=True)
            for c in range(4)
        ]
        out_ref[0] += jnp.concatenate(cols, axis=1)

        new_cnt = cnt + jnp.sum(keep_row).astype(jnp.int32)
        return i + 1, new_cnt

    def outer_cond(carry):
        i, cnt = carry
        return (i < _NT) & (cnt < _N_NMS)

    lax.while_loop(outer_cond, process_tile, (jnp.int32(0), jnp.int32(0)))


def _nms_call(boxes, boxes_t):
    b = boxes.shape[0]
    return pl.pallas_call(
        _nms_body,
        grid=(b,),
        in_specs=[
            pl.BlockSpec((1, _NT, _TILE, 4), lambda bb: (bb, 0, 0, 0)),
            pl.BlockSpec((1, _NT, 4, _TILE), lambda bb: (bb, 0, 0, 0)),
        ],
        out_specs=pl.BlockSpec((1, _OUT_PAD, 4), lambda bb: (bb, 0, 0)),
        out_shape=jax.ShapeDtypeStruct((b, _OUT_PAD, 4), jnp.float32),
        scratch_shapes=[pltpu.VMEM((_NT, 8, _TILE), jnp.float32)],
        compiler_params=pltpu.CompilerParams(
            dimension_semantics=("parallel",)),
    )(boxes, boxes_t)


def kernel(rpn_prob, rpn_roi):
    b = rpn_prob.shape[0]
    n = rpn_prob.shape[1]
    scores = rpn_prob[..., 0]
    bits = lax.bitcast_convert_type(scores, jnp.int32)
    keys = jnp.concatenate(
        [bits, jnp.full((b, _SORT_N - n), -1, jnp.int32)], axis=1)
    sidx = _sort_call(keys.reshape(b, _SORT_R, 128))
    gidx = sidx.reshape(b * _NPAD)
    top_roi = _gather_call(gidx, rpn_roi.reshape(b * n * 4))
    boxes = top_roi.reshape(b, _NT, _TILE, 4)
    boxes_t = jnp.swapaxes(boxes, 2, 3)
    out = _nms_call(boxes, boxes_t)
    return out[:, :_N_NMS, :]
